# Initial kernel scaffold; baseline (speedup 1.0000x reference)
import jax
import jax.numpy as jnp
from jax.experimental import pallas as pl


def kernel(x, edge_index_1, edge_index_2, batch, params):
    raise NotImplementedError("write your pallas kernel here")


# speedup vs baseline: 7.7886x; 7.7886x over previous
import functools

import jax
import jax.numpy as jnp
from jax import lax
from jax.experimental import pallas as pl
from jax.experimental.pallas import tpu as pltpu
from jax.experimental.pallas import tpu_sc as plsc

N = 50000
E = 800000
G = 1000
D = 64
FD = 128
CH = 125
NCH = E // CH
NTILES = 16
CPT = NCH // NTILES
SLAB = 40
NSLAB = CPT // SLAB
NB = 4
ZROWS = 1000
BN_BLK = 2000
N_BLKS = N // BN_BLK

CHP = 100
NCHP_CORE = (N // 2) // CHP


def _agg_phase(y12, e_hbm, acc, srcsl, dstsl, rows, semg, sems, s, plane):
  base = s * CPT
  for si in range(NSLAB):
    row0 = base + si * SLAB
    pltpu.sync_copy(e_hbm.at[0, pl.ds(row0, SLAB)], srcsl)
    pltpu.sync_copy(e_hbm.at[1, pl.ds(row0, SLAB)], dstsl)
    for b in range(NB):
      pltpu.async_copy(y12.at[plane].at[srcsl.at[b]], rows[b], semg[b])

    @pl.loop(0, SLAB // NB)
    def _(g):
      for b in range(NB):
        m = g * NB + b
        pltpu.make_async_copy(
            y12.at[plane].at[srcsl.at[m]], rows[b], semg[b]).wait()
        pltpu.async_copy(rows[b], acc.at[dstsl.at[m]], sems[b], add=True)
      for b in range(NB):
        mn = g * NB + b + NB

        @pl.when(mn < SLAB)
        def _():
          pltpu.make_async_copy(rows[b], acc.at[dstsl.at[0]], sems[b]).wait()
          pltpu.async_copy(y12.at[plane].at[srcsl.at[mn]], rows[b], semg[b])

    for b in range(NB):
      pltpu.make_async_copy(rows[b], acc.at[dstsl.at[0]], sems[b]).wait()


def _agg_body(y12, e1, e2, out, acc, srcsl, dstsl, rows_and_sems):
  rows = rows_and_sems[:NB]
  semg = rows_and_sems[NB:2 * NB]
  sems = rows_and_sems[2 * NB:3 * NB]
  c = lax.axis_index("c")
  s = lax.axis_index("s")

  for ph in range(2):
    e_hbm = e1 if ph == 0 else e2
    for b in range(NB):
      @pl.loop(0, CH * 32 // 16)
      def _(i):
        rows[b][i // 2, pl.ds((i % 2) * 16, 16)] = jnp.zeros(
            (16,), jnp.float32)

    @pl.loop(s, N // CH, step=16)
    def _(q):
      pltpu.sync_copy(rows[0], acc.at[pl.ds(q * CH, CH)])

    plsc.subcore_barrier()
    for cc in range(2):
      @pl.when(c == cc)
      def _():
        _agg_phase(y12, e_hbm, acc, srcsl, dstsl, rows, semg, sems, s,
                   2 * ph + cc)
    plsc.subcore_barrier()
    for cc in range(2):
      @pl.when(c == cc)
      def _():
        @pl.loop(s, N // ZROWS, step=16)
        def _(q):
          pltpu.sync_copy(acc.at[pl.ds(q * ZROWS, ZROWS)],
                          out.at[2 * ph + cc, pl.ds(q * ZROWS, ZROWS)])
    plsc.subcore_barrier()


@functools.partial(jax.jit, donate_argnums=())
def _sc_aggregate(y12, e1r, e2r):
  mesh = plsc.VectorSubcoreMesh(core_axis_name="c", subcore_axis_name="s")
  scratch = [
      pltpu.VMEM_SHARED((N, 32), jnp.float32),
      pltpu.VMEM((SLAB, CH), jnp.int32),
      pltpu.VMEM((SLAB, CH), jnp.int32),
  ]
  scratch += [pltpu.VMEM((CH, 32), jnp.float32) for _ in range(NB)]
  scratch += [pltpu.SemaphoreType.DMA for _ in range(2 * NB)]

  def body(y12_hbm, e1_hbm, e2_hbm, out_hbm, acc, srcsl, dstsl, *rest):
    _agg_body(y12_hbm, e1_hbm, e2_hbm, out_hbm, acc, srcsl, dstsl, rest)

  return pl.kernel(
      body,
      out_type=jax.ShapeDtypeStruct((4, N, 32), jnp.float32),
      mesh=mesh,
      scratch_types=scratch,
      compiler_params=pltpu.CompilerParams(use_tc_tiling_on_sc=False),
  )(y12, e1r, e2r)


def _pool_body(xc, batchr, ones_hbm, outp, outc, accp, accc, bslab, onesb,
               rbuf, zc):
  c = lax.axis_index("c")
  s = lax.axis_index("s")

  pltpu.sync_copy(batchr.at[pl.ds(c * NCHP_CORE, NCHP_CORE)], bslab)
  pltpu.sync_copy(ones_hbm, onesb)

  @pl.loop(0, CHP * D // 16)
  def _(i):
    rbuf[i // 4, pl.ds((i % 4) * 16, 16)] = jnp.zeros((16,), jnp.float32)

  @pl.loop(0, CHP)
  def _(i):
    zc[i, pl.ds(0, 16)] = jnp.zeros((16,), jnp.float32)

  @pl.loop(s, 4 * (G // CHP), step=16)
  def _(q):
    pltpu.sync_copy(rbuf, accp.at[q // 10, pl.ds((q % 10) * CHP, CHP)])

  @pl.loop(s, G // CHP, step=16)
  def _(q):
    pltpu.sync_copy(zc, accc.at[pl.ds(q * CHP, CHP)])

  plsc.subcore_barrier()

  for i in range(4):
    @pl.loop(s, NCHP_CORE, step=16)
    def _(k):
      row0 = c * (N // 2) + k * CHP
      pltpu.sync_copy(xc.at[i, pl.ds(row0, CHP)], rbuf)
      pltpu.sync_copy(rbuf, accp.at[i].at[bslab.at[k]], add=True)
      if i == 0:
        pltpu.sync_copy(onesb, accc.at[bslab.at[k]], add=True)

  plsc.subcore_barrier()

  @pl.loop(s, 4 * (G // CHP), step=16)
  def _(q):
    pltpu.sync_copy(accp.at[q // 10, pl.ds((q % 10) * CHP, CHP)],
                    outp.at[c, q // 10, pl.ds((q % 10) * CHP, CHP)])

  @pl.loop(s, G // CHP, step=16)
  def _(q):
    pltpu.sync_copy(accc.at[pl.ds(q * CHP, CHP)], outc.at[c, pl.ds(q * CHP, CHP)])


@jax.jit
def _sc_pool(xc, batchr, ones_rows):
  mesh = plsc.VectorSubcoreMesh(core_axis_name="c", subcore_axis_name="s")
  scratch = [
      pltpu.VMEM_SHARED((4, G, D), jnp.float32),
      pltpu.VMEM_SHARED((G, 16), jnp.float32),
      pltpu.VMEM((NCHP_CORE, CHP), jnp.int32),
      pltpu.VMEM((CHP, 16), jnp.float32),
      pltpu.VMEM((CHP, D), jnp.float32),
      pltpu.VMEM((CHP, 16), jnp.float32),
  ]
  return pl.kernel(
      _pool_body,
      out_type=(
          jax.ShapeDtypeStruct((2, 4, G, D), jnp.float32),
          jax.ShapeDtypeStruct((2, G, 16), jnp.float32),
      ),
      mesh=mesh,
      scratch_types=scratch,
      compiler_params=pltpu.CompilerParams(use_tc_tiling_on_sc=False),
  )(xc, batchr, ones_rows)


def _write_planes(o_ref, y):
  for p in range(4):
    o_ref[p] = y[:, 32 * p:32 * p + 32]


def _proj1_kernel(x_ref, w_ref, o_ref):
  _write_planes(o_ref, jnp.dot(x_ref[...], w_ref[...],
                               preferred_element_type=jnp.float32))


def _projL_kernel(x_ref, w_ref, s_ref, t_ref, o_ref):
  xa = x_ref[0] * s_ref[...] + t_ref[...]
  _write_planes(o_ref, jnp.dot(xa, w_ref[...],
                               preferred_element_type=jnp.float32))


def _combine_impl(y_ref, a_ref, eps_ref, b1_ref, w21_ref, b21_ref, w22_ref,
                  b22_ref, wm1_ref, bm1_ref, wm2_ref, bm2_ref, xc_ref, st_ref):
  y = y_ref[...]
  ag = a_ref[...]
  eps = eps_ref[...]
  b1 = b1_ref[...]
  z = [jnp.maximum(y[p] * eps[p:p + 1] + ag[p] + b1[p:p + 1], 0.0)
       for p in range(4)]
  w21 = w21_ref[...]
  w22 = w22_ref[...]
  a = jnp.maximum(
      jnp.dot(z[0], w21[:32], preferred_element_type=jnp.float32)
      + jnp.dot(z[1], w21[32:], preferred_element_type=jnp.float32)
      + b21_ref[...], 0.0)
  b = jnp.maximum(
      jnp.dot(z[2], w22[:32], preferred_element_type=jnp.float32)
      + jnp.dot(z[3], w22[32:], preferred_element_type=jnp.float32)
      + b22_ref[...], 0.0)
  wm1 = wm1_ref[...]
  u = jnp.maximum(
      jnp.dot(a, wm1[:D], preferred_element_type=jnp.float32)
      + jnp.dot(b, wm1[D:], preferred_element_type=jnp.float32)
      + bm1_ref[...], 0.0)
  rc = jnp.dot(u, wm2_ref[...], preferred_element_type=jnp.float32) + bm2_ref[...]
  xc_ref[...] = rc[None]
  packed = jnp.stack([jnp.sum(rc, axis=0), jnp.sum(rc * rc, axis=0)])

  @pl.when(pl.program_id(0) == 0)
  def _():
    st_ref[...] = packed

  @pl.when(pl.program_id(0) > 0)
  def _():
    st_ref[...] = st_ref[...] + packed


def _combine0_kernel(y_ref, a_ref, eps_ref, b1_ref, w21_ref, b21_ref, w22_ref,
                     b22_ref, wm1_ref, bm1_ref, wm2_ref, bm2_ref,
                     xc_ref, st_ref):
  _combine_impl(y_ref, a_ref, eps_ref, b1_ref, w21_ref, b21_ref, w22_ref,
                b22_ref, wm1_ref, bm1_ref, wm2_ref, bm2_ref, xc_ref, st_ref)


def _combineL_kernel(y_ref, a_ref, eps_ref, b1_ref, w21_ref, b21_ref, w22_ref,
                     b22_ref, wm1_ref, bm1_ref, wm2_ref, bm2_ref, xcin_ref,
                     xc_ref, st_ref):
  del xcin_ref
  _combine_impl(y_ref, a_ref, eps_ref, b1_ref, w21_ref, b21_ref, w22_ref,
                b22_ref, wm1_ref, bm1_ref, wm2_ref, bm2_ref, xc_ref, st_ref)


def _head_kernel(pp_ref, cp_ref, sf_ref, tf_ref, w1_ref, b1_ref, w2_ref,
                 b2_ref, w3_ref, b3_ref, w4_ref, b4_ref, o_ref):
  pp = pp_ref[...]
  cp = cp_ref[...]
  cnt = (cp[0] + cp[1])[:, 0:1]
  inv = 1.0 / jnp.maximum(cnt, 1.0)
  ind = jnp.minimum(cnt, 1.0)
  sf = sf_ref[...]
  tf = tf_ref[...]
  w1 = w1_ref[...]
  acc = jnp.zeros((G, D), jnp.float32)
  for i in range(4):
    pbi = (pp[0, i] + pp[1, i]) * inv * sf[i][None] + ind * tf[i][None]
    acc = acc + jnp.dot(pbi, w1[i * D:(i + 1) * D],
                        preferred_element_type=jnp.float32)
  h = jnp.maximum(acc + b1_ref[...], 0.0)
  h = jnp.maximum(
      jnp.dot(h, w2_ref[...], preferred_element_type=jnp.float32)
      + b2_ref[...], 0.0)
  h = jnp.maximum(
      jnp.dot(h, w3_ref[...], preferred_element_type=jnp.float32)
      + b3_ref[...], 0.0)
  o = jnp.sum(h * w4_ref[...], axis=1, keepdims=True) + b4_ref[...]
  o_ref[...] = o


def _tc_proj1(x, wf):
  nf = x.shape[1]
  return pl.pallas_call(
      _proj1_kernel,
      grid=(N_BLKS,),
      in_specs=[
          pl.BlockSpec((BN_BLK, nf), lambda i: (i, 0)),
          pl.BlockSpec((nf, FD), lambda i: (0, 0)),
      ],
      out_specs=pl.BlockSpec((4, BN_BLK, 32), lambda i: (0, i, 0)),
      out_shape=jax.ShapeDtypeStruct((4, N, 32), jnp.float32),
  )(x, wf)


def _tc_projL(xc, wf, sv, tv, stripe):
  return pl.pallas_call(
      _projL_kernel,
      grid=(N_BLKS,),
      in_specs=[
          pl.BlockSpec((1, BN_BLK, D), lambda i, s=stripe: (s, i, 0)),
          pl.BlockSpec((D, FD), lambda i: (0, 0)),
          pl.BlockSpec((1, D), lambda i: (0, 0)),
          pl.BlockSpec((1, D), lambda i: (0, 0)),
      ],
      out_specs=pl.BlockSpec((4, BN_BLK, 32), lambda i: (0, i, 0)),
      out_shape=jax.ShapeDtypeStruct((4, N, 32), jnp.float32),
  )(xc, wf, sv, tv)


def _w_specs():
  return [
      pl.BlockSpec((4, 32), lambda i: (0, 0)),
      pl.BlockSpec((4, 32), lambda i: (0, 0)),
      pl.BlockSpec((D, D), lambda i: (0, 0)),
      pl.BlockSpec((1, D), lambda i: (0, 0)),
      pl.BlockSpec((D, D), lambda i: (0, 0)),
      pl.BlockSpec((1, D), lambda i: (0, 0)),
      pl.BlockSpec((FD, D), lambda i: (0, 0)),
      pl.BlockSpec((1, D), lambda i: (0, 0)),
      pl.BlockSpec((D, D), lambda i: (0, 0)),
      pl.BlockSpec((1, D), lambda i: (0, 0)),
  ]


def _tc_combine(y12, agg12, wpack, xc_prev, stripe):
  out_shape = (
      jax.ShapeDtypeStruct((4, N, D), jnp.float32),
      jax.ShapeDtypeStruct((2, D), jnp.float32),
  )
  out_specs = (
      pl.BlockSpec((1, BN_BLK, D), lambda i, s=stripe: (s, i, 0)),
      pl.BlockSpec((2, D), lambda i: (0, 0)),
  )
  data_specs = [
      pl.BlockSpec((4, BN_BLK, 32), lambda i: (0, i, 0)),
      pl.BlockSpec((4, BN_BLK, 32), lambda i: (0, i, 0)),
  ]
  if xc_prev is None:
    return pl.pallas_call(
        _combine0_kernel,
        grid=(N_BLKS,),
        in_specs=data_specs + _w_specs(),
        out_specs=out_specs,
        out_shape=out_shape,
    )(y12, agg12, *wpack)
  return pl.pallas_call(
      _combineL_kernel,
      grid=(N_BLKS,),
      in_specs=data_specs + _w_specs()
      + [pl.BlockSpec(memory_space=pl.ANY)],
      out_specs=out_specs,
      out_shape=out_shape,
      input_output_aliases={12: 0},
  )(y12, agg12, *wpack, xc_prev)


def _tc_head(pooled_p, counts_p, sf, tf, p):
  args = [
      pooled_p, counts_p, sf, tf,
      p["fc1"]["W"], p["fc1"]["b"][None],
      p["fc2"]["W"], p["fc2"]["b"][None],
      p["fc3"]["W"], p["fc3"]["b"][None],
      p["fc4"]["W"].reshape(1, D), p["fc4"]["b"].reshape(1, 1),
  ]
  in_specs = [
      pl.BlockSpec((2, 4, G, D), lambda: (0, 0, 0, 0)),
      pl.BlockSpec((2, G, 16), lambda: (0, 0, 0)),
      pl.BlockSpec((4, D), lambda: (0, 0)),
      pl.BlockSpec((4, D), lambda: (0, 0)),
      pl.BlockSpec((256, D), lambda: (0, 0)),
      pl.BlockSpec((1, D), lambda: (0, 0)),
      pl.BlockSpec((D, D), lambda: (0, 0)),
      pl.BlockSpec((1, D), lambda: (0, 0)),
      pl.BlockSpec((D, D), lambda: (0, 0)),
      pl.BlockSpec((1, D), lambda: (0, 0)),
      pl.BlockSpec((1, D), lambda: (0, 0)),
      pl.BlockSpec((1, 1), lambda: (0, 0)),
  ]
  return pl.pallas_call(
      _head_kernel,
      grid=(),
      in_specs=in_specs,
      out_specs=pl.BlockSpec((G, 1), lambda: (0, 0)),
      out_shape=jax.ShapeDtypeStruct((G, 1), jnp.float32),
  )(*args)


def _layer_wpack(p, i):
  c1 = p["conv%d_1" % i]
  c2 = p["conv%d_2" % i]
  m = p["mlp_%d" % i]
  e1v = jnp.full((32,), 1.0, jnp.float32) + c1["eps"]
  e2v = jnp.full((32,), 1.0, jnp.float32) + c2["eps"]
  epsb = jnp.stack([e1v, e1v, e2v, e2v])
  b11, b12 = c1["l1"]["b"], c2["l1"]["b"]
  b1r = jnp.stack([b11[:32], b11[32:], b12[:32], b12[32:]])
  return (
      epsb, b1r,
      c1["l2"]["W"], c1["l2"]["b"][None],
      c2["l2"]["W"], c2["l2"]["b"][None],
      m["l1"]["W"], m["l1"]["b"][None],
      m["l2"]["W"], m["l2"]["b"][None],
  )


def _fused_w1(p, i):
  return jnp.concatenate(
      [p["conv%d_1" % i]["l1"]["W"], p["conv%d_2" % i]["l1"]["W"]], axis=1)


def kernel(x, edge_index_1, edge_index_2, batch, params):
  e1r = edge_index_1.astype(jnp.int32).reshape(2, NCH, CH)
  e2r = edge_index_2.astype(jnp.int32).reshape(2, NCH, CH)
  batchr = batch.astype(jnp.int32).reshape(N // CHP, CHP)
  ones_rows = jnp.ones((CHP, 16), jnp.float32)

  y12 = _tc_proj1(x, _fused_w1(params, 1))
  xc = None
  svs, tvs = [], []
  for i in range(4):
    agg12 = _sc_aggregate(y12, e1r, e2r)
    xc, stats = _tc_combine(y12, agg12, _layer_wpack(params, i + 1), xc, i)
    bn = params["bn%d" % (i + 1)]
    mean = stats[0] / N
    var = stats[1] / N - mean * mean
    sv = bn["gamma"] / jnp.sqrt(var + 1e-5)
    tv = bn["beta"] - mean * sv
    svs.append(sv)
    tvs.append(tv)
    if i < 3:
      y12 = _tc_projL(xc, _fused_w1(params, i + 2), sv[None], tv[None], i)

  pooled_p, counts_p = _sc_pool(xc, batchr, ones_rows)
  sf = jnp.stack(svs)
  tf = jnp.stack(tvs)
  out = _tc_head(pooled_p, counts_p, sf, tf, params)
  return out.reshape(-1)
